# baseline (speedup 1.0000x reference)
import functools

import jax
import jax.numpy as jnp
from jax import lax
from jax.experimental import pallas as pl
from jax.experimental.pallas import tpu as pltpu
from jax.experimental.pallas import tpu_sc as plsc

H = 256
NCOMB_PAD = 768
SCALE = 16.0

NC = 2
NS = 16
NW = NC * NS
K = 128
RG = 8
BSC = 8


def _prep_body(mw_ref, fw_ref, bw_ref, pw_ref, vw_ref, gw_ref, tab_ref):
    c = lax.broadcasted_iota(jnp.int32, (NCOMB_PAD, H), 0)

    def pick(w_ref, digit):
        w = w_ref[...]
        return jnp.where(digit == 0, w[0:1, :],
                         jnp.where(digit == 1, w[1:2, :], w[2:3, :]))

    acc = pick(mw_ref, c % 3) * SCALE
    acc = acc + pick(fw_ref, (c // 3) % 3) * SCALE
    acc = acc + pick(bw_ref, (c // 9) % 3) * SCALE
    acc = acc + pick(pw_ref, (c // 27) % 3) * SCALE
    acc = acc + pick(vw_ref, (c // 81) % 3) * SCALE
    acc = acc + pick(gw_ref, (c // 243) % 3) * SCALE
    tab_ref[...] = acc


def _make_gather_kernel(B, L):
    P = B * L
    NRG = B // RG
    NCG = NW // NRG
    CW = L // NCG
    PPT = RG * CW
    NCH = PPT // K
    CPB = CW // K
    NBUF = 3
    mesh = plsc.VectorSubcoreMesh(core_axis_name="c", subcore_axis_name="s")

    @functools.partial(
        pl.kernel,
        mesh=mesh,
        out_type=jax.ShapeDtypeStruct((P, H), jnp.float32),
        scratch_types=[
            pltpu.VMEM((6, RG, CW), jnp.int32),
            pltpu.VMEM((NCH, K), jnp.int32),
            pltpu.VMEM((NBUF, K, H), jnp.float32),
            pltpu.SemaphoreType.DMA,
            pltpu.SemaphoreType.DMA,
            pltpu.SemaphoreType.DMA,
            pltpu.SemaphoreType.DMA,
        ],
    )
    def gather_kernel(m_h, f_h, b_h, p_h, v_h, g_h, tab_hbm, out_hbm,
                      idx_v, cidx_v, rows_v, gsem0, gsem1, gsem2, wsem):
        wid = lax.axis_index("s") * NC + lax.axis_index("c")
        rg = wid % NRG
        cg = wid // NRG
        rowbase = pl.multiple_of(rg * RG, 8)
        colbase = pl.multiple_of(cg * CW, 128)

        for i, src in enumerate((m_h, f_h, b_h, p_h, v_h, g_h)):
            pltpu.sync_copy(
                src.at[pl.ds(rowbase, RG), pl.ds(colbase, CW)], idx_v.at[i])

        for j in range(NCH):
            bi, hf = j // CPB, j % CPB
            for k in range(K // 16):
                sl = pl.ds(hf * K + 16 * k, 16)
                cidx_v[j, pl.ds(16 * k, 16)] = (
                    idx_v[0, bi, sl] + 3 * idx_v[1, bi, sl]
                    + 9 * idx_v[2, bi, sl] + 27 * idx_v[3, bi, sl]
                    + 81 * idx_v[4, bi, sl] + 243 * idx_v[5, bi, sl])

        gsems = (gsem0, gsem1, gsem2)

        def gcopy(j):
            return pltpu.make_async_copy(
                tab_hbm.at[cidx_v.at[j]], rows_v.at[j % NBUF],
                gsems[j % NBUF])

        def wcopy(j):
            bi, hf = j // CPB, j % CPB
            off = (rg * RG + bi) * L + cg * CW + hf * K
            return pltpu.make_async_copy(
                rows_v.at[j % NBUF],
                out_hbm.at[pl.ds(pl.multiple_of(off, 8), K)], wsem)

        for j in range(NBUF - 1):
            gcopy(j).start()
        for j in range(NCH):
            if j >= 1:
                wcopy(j - 1).wait()
            if j + NBUF - 1 < NCH:
                gcopy(j + NBUF - 1).start()
            gcopy(j).wait()
            wcopy(j).start()
        wcopy(NCH - 1).wait()

    return gather_kernel


def _select_body(m_ref, f_ref, b_ref, p_ref, v_ref, g_ref,
                 mw_ref, fw_ref, bw_ref, pw_ref, vw_ref, gw_ref, o_ref):
    L = m_ref.shape[-1]
    r3 = lax.broadcasted_iota(jnp.int32, (3, L), 0)
    M = jnp.concatenate(
        [(ref[0] == r3).astype(jnp.float32)
         for ref in (m_ref, f_ref, b_ref, p_ref, v_ref, g_ref)], axis=0)
    W = jnp.concatenate(
        [ref[...] for ref in (mw_ref, fw_ref, bw_ref, pw_ref, vw_ref,
                              gw_ref)], axis=0) * SCALE
    o_ref[...] = lax.dot_general(
        M, W, (((0,), (0,)), ((), ())),
        preferred_element_type=jnp.float32,
        precision=lax.Precision.HIGHEST)


def _tc_select(idx3, ws, B, L, b0):
    nb = B - b0
    grid = (nb,)
    in_specs = (
        [pl.BlockSpec((1, 1, L), lambda g: (g + b0, 0, 0))] * 6
        + [pl.BlockSpec((3, H), lambda g: (0, 0))] * 6)
    out_spec = pl.BlockSpec((L, H), lambda g: (g, 0))
    return pl.pallas_call(
        _select_body,
        grid=grid,
        in_specs=in_specs,
        out_specs=out_spec,
        out_shape=jax.ShapeDtypeStruct((nb * L, H), jnp.float32),
    )(*idx3, *ws)


def kernel(mix, falsetto, breathy, pharyngeal, vibrato, glissando,
           mix_w, falsetto_w, breathy_w, pharyngeal_w, vibrato_w, glissando_w):
    B, L = mix.shape
    idx = [x.astype(jnp.int32)
           for x in (mix, falsetto, breathy, pharyngeal, vibrato, glissando)]
    ws = (mix_w, falsetto_w, breathy_w, pharyngeal_w, vibrato_w, glissando_w)

    tab = pl.pallas_call(
        _prep_body,
        out_shape=jax.ShapeDtypeStruct((NCOMB_PAD, H), jnp.float32),
    )(*ws)

    sc_out = _make_gather_kernel(BSC, L)(*idx, tab)
    idx3 = [x.reshape(B, 1, L) for x in idx]
    tc_out = _tc_select(idx3, ws, B, L, BSC)
    out = jnp.concatenate([sc_out, tc_out], axis=0)
    return out.reshape(B, L, H)


# speedup vs baseline: 1.2379x; 1.2379x over previous
import functools

import jax
import jax.numpy as jnp
from jax import lax
from jax.experimental import pallas as pl
from jax.experimental.pallas import tpu as pltpu
from jax.experimental.pallas import tpu_sc as plsc

H = 256
NCOMB_PAD = 768
SCALE = 16.0

NC = 2
NS = 16
NW = NC * NS
K = 128
RG = 8


def _prep_body(mw_ref, fw_ref, bw_ref, pw_ref, vw_ref, gw_ref, tab_ref):
    c = lax.broadcasted_iota(jnp.int32, (NCOMB_PAD, H), 0)

    def pick(w_ref, digit):
        w = w_ref[...]
        return jnp.where(digit == 0, w[0:1, :],
                         jnp.where(digit == 1, w[1:2, :], w[2:3, :]))

    acc = pick(mw_ref, c % 3) * SCALE
    acc = acc + pick(fw_ref, (c // 3) % 3) * SCALE
    acc = acc + pick(bw_ref, (c // 9) % 3) * SCALE
    acc = acc + pick(pw_ref, (c // 27) % 3) * SCALE
    acc = acc + pick(vw_ref, (c // 81) % 3) * SCALE
    acc = acc + pick(gw_ref, (c // 243) % 3) * SCALE
    tab_ref[...] = acc


def _make_gather_kernel(B, L):
    P = B * L
    NRG = B // RG
    NCG = NW // NRG
    CW = L // NCG
    PPT = RG * CW
    NCH = PPT // K
    CPB = CW // K
    NBUF = 3
    mesh = plsc.VectorSubcoreMesh(core_axis_name="c", subcore_axis_name="s")

    @functools.partial(
        pl.kernel,
        mesh=mesh,
        out_type=jax.ShapeDtypeStruct((P, H), jnp.float32),
        scratch_types=[
            pltpu.VMEM((6, RG, CW), jnp.int32),
            pltpu.VMEM((NCH, K), jnp.int32),
            pltpu.VMEM((NBUF, K, H), jnp.float32),
            pltpu.SemaphoreType.DMA,
            pltpu.SemaphoreType.DMA,
            pltpu.SemaphoreType.DMA,
            pltpu.SemaphoreType.DMA,
        ],
    )
    def gather_kernel(m_h, f_h, b_h, p_h, v_h, g_h, tab_hbm, out_hbm,
                      idx_v, cidx_v, rows_v, gsem0, gsem1, gsem2, wsem):
        wid = lax.axis_index("s") * NC + lax.axis_index("c")
        rg = wid % NRG
        cg = wid // NRG
        rowbase = pl.multiple_of(rg * RG, 8)
        colbase = pl.multiple_of(cg * CW, 128)

        def icopy(i, src):
            return pltpu.make_async_copy(
                src.at[pl.ds(rowbase, RG), pl.ds(colbase, CW)], idx_v.at[i],
                wsem)

        srcs = (m_h, f_h, b_h, p_h, v_h, g_h)
        for i, src in enumerate(srcs):
            icopy(i, src).start()
        for i, src in enumerate(srcs):
            icopy(i, src).wait()

        for j in range(NCH):
            bi, hf = j // CPB, j % CPB
            for k in range(K // 16):
                sl = pl.ds(hf * K + 16 * k, 16)
                cidx_v[j, pl.ds(16 * k, 16)] = (
                    idx_v[0, bi, sl] + 3 * idx_v[1, bi, sl]
                    + 9 * idx_v[2, bi, sl] + 27 * idx_v[3, bi, sl]
                    + 81 * idx_v[4, bi, sl] + 243 * idx_v[5, bi, sl])

        gsems = (gsem0, gsem1, gsem2)

        def gcopy(j):
            return pltpu.make_async_copy(
                tab_hbm.at[cidx_v.at[j]], rows_v.at[j % NBUF],
                gsems[j % NBUF])

        def wcopy(j):
            bi, hf = j // CPB, j % CPB
            off = (rg * RG + bi) * L + cg * CW + hf * K
            return pltpu.make_async_copy(
                rows_v.at[j % NBUF],
                out_hbm.at[pl.ds(pl.multiple_of(off, 8), K)], wsem)

        for j in range(NBUF - 1):
            gcopy(j).start()
        for j in range(NCH):
            if j >= 1:
                wcopy(j - 1).wait()
            if j + NBUF - 1 < NCH:
                gcopy(j + NBUF - 1).start()
            gcopy(j).wait()
            wcopy(j).start()
        wcopy(NCH - 1).wait()

    return gather_kernel


def kernel(mix, falsetto, breathy, pharyngeal, vibrato, glissando,
           mix_w, falsetto_w, breathy_w, pharyngeal_w, vibrato_w, glissando_w):
    B, L = mix.shape
    idx = [x.astype(jnp.int32)
           for x in (mix, falsetto, breathy, pharyngeal, vibrato, glissando)]
    ws = (mix_w, falsetto_w, breathy_w, pharyngeal_w, vibrato_w, glissando_w)

    tab = pl.pallas_call(
        _prep_body,
        out_shape=jax.ShapeDtypeStruct((NCOMB_PAD, H), jnp.float32),
    )(*ws)

    out = _make_gather_kernel(B, L)(*idx, tab)
    return out.reshape(B, L, H)


# speedup vs baseline: 1.2575x; 1.0158x over previous
import functools

import jax
import jax.numpy as jnp
from jax import lax
from jax.experimental import pallas as pl
from jax.experimental.pallas import tpu as pltpu
from jax.experimental.pallas import tpu_sc as plsc

H = 256
NCOMB_PAD = 768
SCALE = 16.0

NC = 2
NS = 16
NW = NC * NS
K = 128
RG = 8


def _prep_body(mw_ref, fw_ref, bw_ref, pw_ref, vw_ref, gw_ref, tab_ref):
    c = lax.broadcasted_iota(jnp.int32, (NCOMB_PAD, H), 0)

    def pick(w_ref, digit):
        w = w_ref[...]
        return jnp.where(digit == 0, w[0:1, :],
                         jnp.where(digit == 1, w[1:2, :], w[2:3, :]))

    acc = pick(mw_ref, c % 3) * SCALE
    acc = acc + pick(fw_ref, (c // 3) % 3) * SCALE
    acc = acc + pick(bw_ref, (c // 9) % 3) * SCALE
    acc = acc + pick(pw_ref, (c // 27) % 3) * SCALE
    acc = acc + pick(vw_ref, (c // 81) % 3) * SCALE
    acc = acc + pick(gw_ref, (c // 243) % 3) * SCALE
    tab_ref[...] = acc


def _make_gather_kernel(B, L):
    P = B * L
    NRG = B // RG
    NCG = NW // NRG
    CW = L // NCG
    PPT = RG * CW
    NCH = PPT // K
    CPB = CW // K
    NBUF = 3
    mesh = plsc.VectorSubcoreMesh(core_axis_name="c", subcore_axis_name="s")

    @functools.partial(
        pl.kernel,
        mesh=mesh,
        out_type=jax.ShapeDtypeStruct((P, H), jnp.float32),
        scratch_types=[
            pltpu.VMEM((6, RG, CW), jnp.int32),
            pltpu.VMEM((NCH, K), jnp.int32),
            pltpu.VMEM((NBUF, K, H), jnp.float32),
            pltpu.SemaphoreType.DMA,
            pltpu.SemaphoreType.DMA,
            pltpu.SemaphoreType.DMA,
            pltpu.SemaphoreType.DMA,
        ],
    )
    def gather_kernel(m_h, f_h, b_h, p_h, v_h, g_h, tab_hbm, out_hbm,
                      idx_v, cidx_v, rows_v, gsem0, gsem1, gsem2, wsem):
        wid = lax.axis_index("s") * NC + lax.axis_index("c")
        rg = wid % NRG
        cg = wid // NRG
        rowbase = pl.multiple_of(rg * RG, 8)
        colbase = pl.multiple_of(cg * CW, 128)

        def icopy(i, src):
            return pltpu.make_async_copy(
                src.at[pl.ds(rowbase, RG), pl.ds(colbase, CW)], idx_v.at[i],
                wsem)

        srcs = (m_h, f_h, b_h, p_h, v_h, g_h)
        for i, src in enumerate(srcs):
            icopy(i, src).start()
        for i, src in enumerate(srcs):
            icopy(i, src).wait()

        def fuse_chunk(j):
            bi, hf = j // CPB, j % CPB
            for k in range(K // 16):
                sl = pl.ds(hf * K + 16 * k, 16)
                cidx_v[j, pl.ds(16 * k, 16)] = (
                    idx_v[0, bi, sl] + 3 * idx_v[1, bi, sl]
                    + 9 * idx_v[2, bi, sl] + 27 * idx_v[3, bi, sl]
                    + 81 * idx_v[4, bi, sl] + 243 * idx_v[5, bi, sl])

        gsems = (gsem0, gsem1, gsem2)

        def gcopy(j):
            return pltpu.make_async_copy(
                tab_hbm.at[cidx_v.at[j]], rows_v.at[j % NBUF],
                gsems[j % NBUF])

        def wcopy(j):
            bi, hf = j // CPB, j % CPB
            off = (rg * RG + bi) * L + cg * CW + hf * K
            return pltpu.make_async_copy(
                rows_v.at[j % NBUF],
                out_hbm.at[pl.ds(pl.multiple_of(off, 8), K)], wsem)

        for j in range(NBUF - 1):
            fuse_chunk(j)
            gcopy(j).start()
        for j in range(NBUF - 1, NCH):
            fuse_chunk(j)
        for j in range(NCH):
            if j >= 1:
                wcopy(j - 1).wait()
            if j + NBUF - 1 < NCH:
                gcopy(j + NBUF - 1).start()
            gcopy(j).wait()
            wcopy(j).start()
        wcopy(NCH - 1).wait()

    return gather_kernel


def kernel(mix, falsetto, breathy, pharyngeal, vibrato, glissando,
           mix_w, falsetto_w, breathy_w, pharyngeal_w, vibrato_w, glissando_w):
    B, L = mix.shape
    idx = [x.astype(jnp.int32)
           for x in (mix, falsetto, breathy, pharyngeal, vibrato, glissando)]
    ws = (mix_w, falsetto_w, breathy_w, pharyngeal_w, vibrato_w, glissando_w)

    tab = pl.pallas_call(
        _prep_body,
        out_shape=jax.ShapeDtypeStruct((NCOMB_PAD, H), jnp.float32),
    )(*ws)

    out = _make_gather_kernel(B, L)(*idx, tab)
    return out.reshape(B, L, H)
